# Initial kernel scaffold; baseline (speedup 1.0000x reference)
import jax
import jax.numpy as jnp
from jax.experimental import pallas as pl


def kernel(x, d, hash_tables, xW0, xb0, xW1, xb1, dW0, db0, dW1, db1, dW2, db2):
    raise NotImplementedError("write your pallas kernel here")


# speedup vs baseline: 2.2011x; 2.2011x over previous
import functools
import math

import jax
import jax.numpy as jnp
import numpy as np
from jax import lax
from jax.experimental import pallas as pl
from jax.experimental.pallas import tpu as pltpu
from jax.experimental.pallas import tpu_sc as plsc

N_LEVELS = 16
F = 2
T = 1 << 19
BASE = 16.0
PLS = math.exp((math.log(2048.0) - math.log(16.0)) / (N_LEVELS - 1))
P1 = np.uint32(2654435761).view(np.int32).item()
P2 = np.uint32(805459861).view(np.int32).item()
AABB_LO = np.array([[-1.0, -1.0, -1.0]], dtype=np.float32)
AABB_HI = np.array([[1.0, 1.0, 1.0]], dtype=np.float32)

_LVL = []
for _l in range(N_LEVELS):
    _s = BASE * (PLS ** _l) - 1.0
    _res = int(math.ceil(_s)) + 1
    _LVL.append((np.float32(_s), _res, (_res ** 3) <= T))

NW = 32
C = 512
SUB = 128
NSTR = (8 * C) // SUB


def _make_encoder(n_pts):
    pts_per_w = n_pts // NW
    nch = pts_per_w // C
    mesh = plsc.VectorSubcoreMesh(core_axis_name="c", subcore_axis_name="s")

    @functools.partial(
        pl.kernel,
        mesh=mesh,
        out_type=jax.ShapeDtypeStruct((n_pts * 32,), jnp.float32),
        scratch_types=[
            pltpu.VMEM((3 * C,), jnp.float32),
            pltpu.VMEM((2, 1, 3 * C), jnp.float32),
            pltpu.VMEM((2, 1, 8 * C), jnp.int32),
            pltpu.VMEM((2, 1, 8 * C), jnp.int32),
            pltpu.VMEM((C * 32,), jnp.float32),
            pltpu.SemaphoreType.DMA,
            pltpu.SemaphoreType.DMA,
        ],
    )
    def encode(xct_hbm, ptab_hbm, out_hbm, xbuf, wbuf, idxbuf, gbuf, fbuf,
               sem0, sem1):
        wid = lax.axis_index("s") * 2 + lax.axis_index("c")
        base = wid * pts_per_w
        sems = (sem0, sem1)

        def idx_pass(l, b):
            s_f, res, dense = _LVL[l]
            lbase = l * T

            def body(g, carry):
                p0 = g * 16
                xv = xbuf[pl.ds(p0, 16)]
                yv = xbuf[pl.ds(C + p0, 16)]
                zv = xbuf[pl.ds(2 * C + p0, 16)]
                px = xv * s_f + np.float32(0.5)
                py = yv * s_f + np.float32(0.5)
                pz = zv * s_f + np.float32(0.5)
                ix = px.astype(jnp.int32)
                iy = py.astype(jnp.int32)
                iz = pz.astype(jnp.int32)
                wbuf[b, 0, pl.ds(p0, 16)] = px - ix.astype(jnp.float32)
                wbuf[b, 0, pl.ds(C + p0, 16)] = py - iy.astype(jnp.float32)
                wbuf[b, 0, pl.ds(2 * C + p0, 16)] = pz - iz.astype(jnp.float32)
                if dense:
                    bx0 = ix
                    bx1 = ix + 1
                    by0 = iy * np.int32(res)
                    by1 = by0 + np.int32(res)
                    bz0 = iz * np.int32(res * res)
                    bz1 = bz0 + np.int32(res * res)
                else:
                    bx0 = ix
                    bx1 = ix + 1
                    by0 = iy * np.int32(P1)
                    by1 = by0 + np.int32(P1)
                    bz0 = iz * np.int32(P2)
                    bz1 = bz0 + np.int32(P2)
                bxs = (bx0, bx1)
                bys = (by0, by1)
                bzs = (bz0, bz1)
                for c in range(8):
                    dx, dy, dz = c & 1, (c >> 1) & 1, (c >> 2) & 1
                    if dense:
                        h = bxs[dx] + bys[dy] + bzs[dz]
                    else:
                        h = bxs[dx] ^ bys[dy] ^ bzs[dz]
                    idxbuf[b, 0, pl.ds(c * C + p0, 16)] = (
                        (h & np.int32(T - 1)) + np.int32(lbase))
                return carry

            lax.fori_loop(0, C // 16, body, 0, unroll=False)

        def fire(l, b):
            return [
                pltpu.async_copy(
                    ptab_hbm.at[idxbuf.at[b, 0, pl.ds(s * SUB, SUB)]],
                    gbuf.at[b, 0, pl.ds(s * SUB, SUB)],
                    sems[b],
                )
                for s in range(NSTR)
            ]

        def mac_pass(l, b):
            def body(g, carry):
                p0 = g * 16
                wx = wbuf[b, 0, pl.ds(p0, 16)]
                wy = wbuf[b, 0, pl.ds(C + p0, 16)]
                wz = wbuf[b, 0, pl.ds(2 * C + p0, 16)]
                one = np.float32(1.0)
                ux = one - wx
                uy = one - wy
                uz = one - wz
                a = ((ux * uy, wx * uy), (ux * wy, wx * wy))
                zcs = (uz, wz)
                acc0 = jnp.zeros((16,), jnp.float32)
                acc1 = jnp.zeros((16,), jnp.float32)
                for c in range(8):
                    dx, dy, dz = c & 1, (c >> 1) & 1, (c >> 2) & 1
                    wc = a[dy][dx] * zcs[dz]
                    word = gbuf[b, 0, pl.ds(c * C + p0, 16)]
                    f0c = lax.bitcast_convert_type(word << 16, jnp.float32)
                    f1c = lax.bitcast_convert_type(word & np.int32(-65536),
                                                   jnp.float32)
                    acc0 = acc0 + wc * f0c
                    acc1 = acc1 + wc * f1c
                fbuf[pl.ds((2 * l) * C + p0, 16)] = acc0
                fbuf[pl.ds((2 * l + 1) * C + p0, 16)] = acc1
                return carry

            lax.fori_loop(0, C // 16, body, 0, unroll=False)

        def chunk_body(ch, carry):
            row0 = base + ch * C
            for dim in range(3):
                pltpu.sync_copy(xct_hbm.at[pl.ds(dim * n_pts + row0, C)],
                                xbuf.at[pl.ds(dim * C, C)])
            idx_pass(0, 0)
            pending = {0: fire(0, 0)}
            for l in range(N_LEVELS):
                if l + 1 < N_LEVELS:
                    idx_pass(l + 1, (l + 1) % 2)
                    pending[l + 1] = fire(l + 1, (l + 1) % 2)
                for h in pending.pop(l):
                    h.wait()
                mac_pass(l, l % 2)
            pltpu.sync_copy(fbuf, out_hbm.at[pl.ds(row0 * 32, C * 32)])
            return carry

        lax.fori_loop(0, nch, chunk_body, 0, unroll=False)

    return encode


BN = 2048


def _sh16(dv):
    x = dv[:, 0] * 2.0 - 1.0
    y = dv[:, 1] * 2.0 - 1.0
    z = dv[:, 2] * 2.0 - 1.0
    xy = x * y
    xz = x * z
    yz = y * z
    x2 = x * x
    y2 = y * y
    z2 = z * z
    return jnp.stack([
        0.28209479177387814 * jnp.ones_like(x),
        -0.48860251190291987 * y,
        0.48860251190291987 * z,
        -0.48860251190291987 * x,
        1.0925484305920792 * xy,
        -1.0925484305920792 * yz,
        0.94617469575755997 * z2 - 0.31539156525251999,
        -1.0925484305920792 * xz,
        0.54627421529603959 * (x2 - y2),
        0.59004358992664352 * y * (-3.0 * x2 + y2),
        2.8906114426405538 * xy * z,
        0.45704579946446572 * y * (1.0 - 5.0 * z2),
        0.3731763325901154 * z * (5.0 * z2 - 3.0),
        0.45704579946446572 * x * (1.0 - 5.0 * z2),
        1.4453057213202769 * z * (x2 - y2),
        0.59004358992664352 * x * (-x2 + 3.0 * y2),
    ], axis=-1)


def _mlp_body(feats_ref, d_ref, xw0_ref, xb0_ref, xw1a_ref, xb1a_ref,
              xw1b_ref, xb1b_ref, dw0_ref, db0_ref, dw1_ref, db1_ref,
              dw2_ref, db2_ref, sigma_ref, rgb_ref):
    feats = feats_ref[...]
    h = jnp.maximum(
        jnp.dot(feats, xw0_ref[...], preferred_element_type=jnp.float32)
        + xb0_ref[...], 0.0)
    f0 = (jnp.dot(h, xw1a_ref[...], preferred_element_type=jnp.float32)[:, 0]
          + xb1a_ref[0])
    frest = (jnp.dot(h, xw1b_ref[...], preferred_element_type=jnp.float32)
             + xb1b_ref[...])
    sigma_ref[...] = jnp.log1p(jnp.exp(-jnp.abs(f0))) + jnp.maximum(f0, 0.0)
    dv = d_ref[...] * 0.5 + 0.5
    sh = _sh16(dv)
    hd = jnp.concatenate([sh, frest], axis=-1)
    h1 = jnp.maximum(
        jnp.dot(hd, dw0_ref[...], preferred_element_type=jnp.float32)
        + db0_ref[...], 0.0)
    h2 = jnp.maximum(
        jnp.dot(h1, dw1_ref[...], preferred_element_type=jnp.float32)
        + db1_ref[...], 0.0)
    out = (jnp.dot(h2, dw2_ref[...], preferred_element_type=jnp.float32)
           + db2_ref[...])
    rgb_ref[...] = jax.nn.sigmoid(out)


def _full_spec(shape):
    nd = len(shape)
    return pl.BlockSpec(shape, lambda i: (0,) * nd)


def _mlp_call(feats, d, xW0, xb0, xW1a, xb1a, xW1b, xb1b, dW0, db0, dW1, db1,
              dW2, db2):
    n = feats.shape[0]
    grid = (n // BN,)
    return pl.pallas_call(
        _mlp_body,
        grid=grid,
        in_specs=[
            pl.BlockSpec((BN, 32), lambda i: (i, 0)),
            pl.BlockSpec((BN, 3), lambda i: (i, 0)),
            _full_spec(xW0.shape),
            _full_spec(xb0.shape),
            _full_spec(xW1a.shape),
            _full_spec(xb1a.shape),
            _full_spec(xW1b.shape),
            _full_spec(xb1b.shape),
            _full_spec(dW0.shape),
            _full_spec(db0.shape),
            _full_spec(dW1.shape),
            _full_spec(db1.shape),
            _full_spec(dW2.shape),
            _full_spec(db2.shape),
        ],
        out_specs=[
            pl.BlockSpec((BN,), lambda i: (i,)),
            pl.BlockSpec((BN, 3), lambda i: (i, 0)),
        ],
        out_shape=[
            jax.ShapeDtypeStruct((n,), jnp.float32),
            jax.ShapeDtypeStruct((n, 3), jnp.float32),
        ],
    )(feats, d, xW0, xb0, xW1a, xb1a, xW1b, xb1b, dW0, db0, dW1, db1, dW2,
      db2)


def kernel(x, d, hash_tables, xW0, xb0, xW1, xb1, dW0, db0, dW1, db1, dW2,
           db2):
    n = x.shape[0]
    xn = (x - AABB_LO) / (AABB_HI - AABB_LO) * 2.0 - 1.0
    xc = xn / 4.0 + 0.5
    xct = xc.T.reshape(-1)
    ptab = lax.bitcast_convert_type(
        hash_tables.astype(jnp.bfloat16), jnp.int32).reshape(-1)
    feats_cm = _make_encoder(n)(xct, ptab)
    feats = feats_cm.reshape(n // C, 32, C).transpose(0, 2, 1).reshape(n, 32)
    xW1a = xW1[:, 0:1]
    xb1a = xb1[0:1]
    xW1b = xW1[:, 1:]
    xb1b = xb1[1:]
    sigma, rgb = _mlp_call(feats, d, xW0, xb0, xW1a, xb1a, xW1b, xb1b, dW0,
                           db0, dW1, db1, dW2, db2)
    return sigma, rgb


# speedup vs baseline: 2.6476x; 1.2029x over previous
import functools
import math

import jax
import jax.numpy as jnp
import numpy as np
from jax import lax
from jax.experimental import pallas as pl
from jax.experimental.pallas import tpu as pltpu
from jax.experimental.pallas import tpu_sc as plsc

N_LEVELS = 16
F = 2
T = 1 << 19
BASE = 16.0
PLS = math.exp((math.log(2048.0) - math.log(16.0)) / (N_LEVELS - 1))
P1 = np.uint32(2654435761).view(np.int32).item()
P2 = np.uint32(805459861).view(np.int32).item()
AABB_LO = np.array([[-1.0, -1.0, -1.0]], dtype=np.float32)
AABB_HI = np.array([[1.0, 1.0, 1.0]], dtype=np.float32)

_LVL = []
for _l in range(N_LEVELS):
    _s = BASE * (PLS ** _l) - 1.0
    _res = int(math.ceil(_s)) + 1
    _LVL.append((np.float32(_s), _res, (_res ** 3) <= T))

NW = 32
C = 512
SUB = 128
NSTR = (8 * C) // SUB


def _make_encoder(n_pts):
    pts_per_w = n_pts // NW
    nch = pts_per_w // C
    mesh = plsc.VectorSubcoreMesh(core_axis_name="c", subcore_axis_name="s")

    @functools.partial(
        pl.kernel,
        mesh=mesh,
        out_type=jax.ShapeDtypeStruct((n_pts * 32,), jnp.float32),
        scratch_types=[
            pltpu.VMEM((3 * C,), jnp.float32),
            pltpu.VMEM((2, 1, 3 * C), jnp.float32),
            pltpu.VMEM((2, 1, 8 * C), jnp.int32),
            pltpu.VMEM((2, 1, 8 * C), jnp.int32),
            pltpu.VMEM((C * 32,), jnp.float32),
            pltpu.SemaphoreType.DMA,
            pltpu.SemaphoreType.DMA,
        ],
    )
    def encode(xct_hbm, ptab_hbm, out_hbm, xbuf, wbuf, idxbuf, gbuf, fbuf,
               sem0, sem1):
        wid = lax.axis_index("s") * 2 + lax.axis_index("c")
        base = wid * pts_per_w
        sems = (sem0, sem1)

        def idx_pass(l, b):
            s_f, res, dense = _LVL[l]
            lbase = l * T

            def body(g, carry):
                p0 = g * 16
                xv = xbuf[pl.ds(p0, 16)]
                yv = xbuf[pl.ds(C + p0, 16)]
                zv = xbuf[pl.ds(2 * C + p0, 16)]
                px = xv * s_f + np.float32(0.5)
                py = yv * s_f + np.float32(0.5)
                pz = zv * s_f + np.float32(0.5)
                ix = px.astype(jnp.int32)
                iy = py.astype(jnp.int32)
                iz = pz.astype(jnp.int32)
                wbuf[b, 0, pl.ds(p0, 16)] = px - ix.astype(jnp.float32)
                wbuf[b, 0, pl.ds(C + p0, 16)] = py - iy.astype(jnp.float32)
                wbuf[b, 0, pl.ds(2 * C + p0, 16)] = pz - iz.astype(jnp.float32)
                if dense:
                    bx0 = ix
                    bx1 = ix + 1
                    by0 = iy * np.int32(res)
                    by1 = by0 + np.int32(res)
                    bz0 = iz * np.int32(res * res)
                    bz1 = bz0 + np.int32(res * res)
                else:
                    bx0 = ix
                    bx1 = ix + 1
                    by0 = iy * np.int32(P1)
                    by1 = by0 + np.int32(P1)
                    bz0 = iz * np.int32(P2)
                    bz1 = bz0 + np.int32(P2)
                bxs = (bx0, bx1)
                bys = (by0, by1)
                bzs = (bz0, bz1)
                for c in range(8):
                    dx, dy, dz = c & 1, (c >> 1) & 1, (c >> 2) & 1
                    if dense:
                        h = bxs[dx] + bys[dy] + bzs[dz]
                    else:
                        h = bxs[dx] ^ bys[dy] ^ bzs[dz]
                    idxbuf[b, 0, pl.ds(c * C + p0, 16)] = (
                        (h & np.int32(T - 1)) + np.int32(lbase))
                return carry

            lax.fori_loop(0, C // 16, body, 0, unroll=False)

        def fire(l, b):
            return [
                pltpu.async_copy(
                    ptab_hbm.at[idxbuf.at[b, 0, pl.ds(s * SUB, SUB)]],
                    gbuf.at[b, 0, pl.ds(s * SUB, SUB)],
                    sems[b],
                )
                for s in range(NSTR)
            ]

        def mac_pass(l, b):
            def body(g, carry):
                p0 = g * 16
                wx = wbuf[b, 0, pl.ds(p0, 16)]
                wy = wbuf[b, 0, pl.ds(C + p0, 16)]
                wz = wbuf[b, 0, pl.ds(2 * C + p0, 16)]
                one = np.float32(1.0)
                ux = one - wx
                uy = one - wy
                uz = one - wz
                a = ((ux * uy, wx * uy), (ux * wy, wx * wy))
                zcs = (uz, wz)
                acc0 = jnp.zeros((16,), jnp.float32)
                acc1 = jnp.zeros((16,), jnp.float32)
                for c in range(8):
                    dx, dy, dz = c & 1, (c >> 1) & 1, (c >> 2) & 1
                    wc = a[dy][dx] * zcs[dz]
                    word = gbuf[b, 0, pl.ds(c * C + p0, 16)]
                    f0c = lax.bitcast_convert_type(word << 16, jnp.float32)
                    f1c = lax.bitcast_convert_type(word & np.int32(-65536),
                                                   jnp.float32)
                    acc0 = acc0 + wc * f0c
                    acc1 = acc1 + wc * f1c
                fbuf[pl.ds((2 * l) * C + p0, 16)] = acc0
                fbuf[pl.ds((2 * l + 1) * C + p0, 16)] = acc1
                return carry

            lax.fori_loop(0, C // 16, body, 0, unroll=False)

        def chunk_body(ch, carry):
            row0 = base + ch * C
            for dim in range(3):
                pltpu.sync_copy(xct_hbm.at[pl.ds(dim * n_pts + row0, C)],
                                xbuf.at[pl.ds(dim * C, C)])
            idx_pass(0, 0)
            pending = {0: fire(0, 0)}
            for l in range(N_LEVELS):
                if l + 1 < N_LEVELS:
                    idx_pass(l + 1, (l + 1) % 2)
                    pending[l + 1] = fire(l + 1, (l + 1) % 2)
                for h in pending.pop(l):
                    h.wait()
                mac_pass(l, l % 2)
            pltpu.sync_copy(fbuf, out_hbm.at[pl.ds(row0 * 32, C * 32)])
            return carry

        lax.fori_loop(0, nch, chunk_body, 0, unroll=False)

    return encode


BN = 2048


def _sh16_rows(x, y, z):
    xy = x * y
    xz = x * z
    yz = y * z
    x2 = x * x
    y2 = y * y
    z2 = z * z
    return jnp.concatenate([
        0.28209479177387814 * jnp.ones_like(x),
        -0.48860251190291987 * y,
        0.48860251190291987 * z,
        -0.48860251190291987 * x,
        1.0925484305920792 * xy,
        -1.0925484305920792 * yz,
        0.94617469575755997 * z2 - 0.31539156525251999,
        -1.0925484305920792 * xz,
        0.54627421529603959 * (x2 - y2),
        0.59004358992664352 * y * (-3.0 * x2 + y2),
        2.8906114426405538 * xy * z,
        0.45704579946446572 * y * (1.0 - 5.0 * z2),
        0.3731763325901154 * z * (5.0 * z2 - 3.0),
        0.45704579946446572 * x * (1.0 - 5.0 * z2),
        1.4453057213202769 * z * (x2 - y2),
        0.59004358992664352 * x * (-x2 + 3.0 * y2),
    ], axis=0)


def _mlp_body(feats_ref, dt_ref, xw0t_ref, xb0c_ref, xw1at_ref, xb1ac_ref,
              xw1bt_ref, xb1bc_ref, dw0t_ref, db0c_ref, dw1t_ref, db1c_ref,
              dw2t_ref, db2c_ref, sigma_ref, rgbt_ref):
    feats = feats_ref[...]
    h = jnp.maximum(
        jnp.dot(xw0t_ref[...], feats, preferred_element_type=jnp.float32)
        + xb0c_ref[...], 0.0)
    f0 = (jnp.dot(xw1at_ref[...], h, preferred_element_type=jnp.float32)
          + xb1ac_ref[...])
    frest = (jnp.dot(xw1bt_ref[...], h, preferred_element_type=jnp.float32)
             + xb1bc_ref[...])
    sigma_ref[...] = jnp.log1p(jnp.exp(-jnp.abs(f0))) + jnp.maximum(f0, 0.0)
    dv = dt_ref[...]
    u = dv * 0.5 + 0.5
    v = u * 2.0 - 1.0
    sh = _sh16_rows(v[0:1, :], v[1:2, :], v[2:3, :])
    hd = jnp.concatenate([sh, frest], axis=0)
    h1 = jnp.maximum(
        jnp.dot(dw0t_ref[...], hd, preferred_element_type=jnp.float32)
        + db0c_ref[...], 0.0)
    h2 = jnp.maximum(
        jnp.dot(dw1t_ref[...], h1, preferred_element_type=jnp.float32)
        + db1c_ref[...], 0.0)
    out = (jnp.dot(dw2t_ref[...], h2, preferred_element_type=jnp.float32)
           + db2c_ref[...])
    rgbt_ref[...] = jax.nn.sigmoid(out)


def _full_spec(shape):
    nd = len(shape)
    return pl.BlockSpec(shape, lambda i: (0,) * nd)


def _mlp_call(feats_t, d_t, *weights):
    n = feats_t.shape[1]
    grid = (n // BN,)
    return pl.pallas_call(
        _mlp_body,
        grid=grid,
        in_specs=[
            pl.BlockSpec((32, BN), lambda i: (0, i)),
            pl.BlockSpec((3, BN), lambda i: (0, i)),
        ] + [_full_spec(w.shape) for w in weights],
        out_specs=[
            pl.BlockSpec((1, BN), lambda i: (0, i)),
            pl.BlockSpec((3, BN), lambda i: (0, i)),
        ],
        out_shape=[
            jax.ShapeDtypeStruct((1, n), jnp.float32),
            jax.ShapeDtypeStruct((3, n), jnp.float32),
        ],
    )(feats_t, d_t, *weights)


def kernel(x, d, hash_tables, xW0, xb0, xW1, xb1, dW0, db0, dW1, db1, dW2,
           db2):
    n = x.shape[0]
    xn = (x - AABB_LO) / (AABB_HI - AABB_LO) * 2.0 - 1.0
    xc = xn / 4.0 + 0.5
    xct = xc.T.reshape(-1)
    ptab = lax.bitcast_convert_type(
        hash_tables.astype(jnp.bfloat16), jnp.int32).reshape(-1)
    feats_cm = _make_encoder(n)(xct, ptab)
    feats_t = feats_cm.reshape(n // C, 32, C).transpose(1, 0, 2).reshape(32, n)
    sigma2d, rgbt = _mlp_call(
        feats_t, d.T,
        xW0.T, xb0[:, None],
        xW1[:, 0:1].T, xb1[0:1][:, None],
        xW1[:, 1:].T, xb1[1:][:, None],
        dW0.T, db0[:, None],
        dW1.T, db1[:, None],
        dW2.T, db2[:, None],
    )
    return sigma2d.reshape(n), rgbt.T


# speedup vs baseline: 2.6908x; 1.0163x over previous
import functools
import math

import jax
import jax.numpy as jnp
import numpy as np
from jax import lax
from jax.experimental import pallas as pl
from jax.experimental.pallas import tpu as pltpu
from jax.experimental.pallas import tpu_sc as plsc

N_LEVELS = 16
F = 2
T = 1 << 19
BASE = 16.0
PLS = math.exp((math.log(2048.0) - math.log(16.0)) / (N_LEVELS - 1))
P1 = np.uint32(2654435761).view(np.int32).item()
P2 = np.uint32(805459861).view(np.int32).item()
AABB_LO = np.array([[-1.0, -1.0, -1.0]], dtype=np.float32)
AABB_HI = np.array([[1.0, 1.0, 1.0]], dtype=np.float32)

_LVL = []
for _l in range(N_LEVELS):
    _s = BASE * (PLS ** _l) - 1.0
    _res = int(math.ceil(_s)) + 1
    _LVL.append((np.float32(_s), _res, (_res ** 3) <= T))

NW = 32
C = 512
SUB = 128
NSTR = (8 * C) // SUB


def _make_encoder(n_pts):
    pts_per_w = n_pts // NW
    nch = pts_per_w // C
    mesh = plsc.VectorSubcoreMesh(core_axis_name="c", subcore_axis_name="s")

    @functools.partial(
        pl.kernel,
        mesh=mesh,
        out_type=jax.ShapeDtypeStruct((n_pts * 32,), jnp.float32),
        scratch_types=[
            pltpu.VMEM((3 * C,), jnp.float32),
            pltpu.VMEM((2, 1, 3 * C), jnp.float32),
            pltpu.VMEM((2, 1, 8 * C), jnp.int32),
            pltpu.VMEM((2, 1, 8 * C), jnp.int32),
            pltpu.VMEM((C * 32,), jnp.float32),
            pltpu.SemaphoreType.DMA,
            pltpu.SemaphoreType.DMA,
        ],
    )
    def encode(xct_hbm, ptab_hbm, out_hbm, xbuf, wbuf, idxbuf, gbuf, fbuf,
               sem0, sem1):
        wid = lax.axis_index("s") * 2 + lax.axis_index("c")
        base = wid * pts_per_w
        sems = (sem0, sem1)

        def idx_pass(l, b):
            s_f, res, dense = _LVL[l]
            lbase = l * T

            def body(g, carry):
                p0 = g * 16
                xv = xbuf[pl.ds(p0, 16)]
                yv = xbuf[pl.ds(C + p0, 16)]
                zv = xbuf[pl.ds(2 * C + p0, 16)]
                px = xv * s_f + np.float32(0.5)
                py = yv * s_f + np.float32(0.5)
                pz = zv * s_f + np.float32(0.5)
                ix = px.astype(jnp.int32)
                iy = py.astype(jnp.int32)
                iz = pz.astype(jnp.int32)
                wbuf[b, 0, pl.ds(p0, 16)] = px - ix.astype(jnp.float32)
                wbuf[b, 0, pl.ds(C + p0, 16)] = py - iy.astype(jnp.float32)
                wbuf[b, 0, pl.ds(2 * C + p0, 16)] = pz - iz.astype(jnp.float32)
                if dense:
                    bx0 = ix
                    bx1 = ix + 1
                    by0 = iy * np.int32(res)
                    by1 = by0 + np.int32(res)
                    bz0 = iz * np.int32(res * res)
                    bz1 = bz0 + np.int32(res * res)
                else:
                    bx0 = ix
                    bx1 = ix + 1
                    by0 = iy * np.int32(P1)
                    by1 = by0 + np.int32(P1)
                    bz0 = iz * np.int32(P2)
                    bz1 = bz0 + np.int32(P2)
                bxs = (bx0, bx1)
                bys = (by0, by1)
                bzs = (bz0, bz1)
                for c in range(8):
                    dx, dy, dz = c & 1, (c >> 1) & 1, (c >> 2) & 1
                    if dense:
                        h = bxs[dx] + bys[dy] + bzs[dz]
                    else:
                        h = bxs[dx] ^ bys[dy] ^ bzs[dz]
                    idxbuf[b, 0, pl.ds(p0 * 8 + c * 16, 16)] = (
                        (h & np.int32(T - 1)) + np.int32(lbase))
                return carry

            lax.fori_loop(0, C // 16, body, 0, unroll=False)

        def fire(l, b):
            return [
                pltpu.async_copy(
                    ptab_hbm.at[idxbuf.at[b, 0]],
                    gbuf.at[b, 0],
                    sems[b],
                )
            ]

        def mac_pass(l, b):
            def body(g, carry):
                p0 = g * 16
                wx = wbuf[b, 0, pl.ds(p0, 16)]
                wy = wbuf[b, 0, pl.ds(C + p0, 16)]
                wz = wbuf[b, 0, pl.ds(2 * C + p0, 16)]
                one = np.float32(1.0)
                ux = one - wx
                uy = one - wy
                uz = one - wz
                a = ((ux * uy, wx * uy), (ux * wy, wx * wy))
                zcs = (uz, wz)
                acc0 = jnp.zeros((16,), jnp.float32)
                acc1 = jnp.zeros((16,), jnp.float32)
                for c in range(8):
                    dx, dy, dz = c & 1, (c >> 1) & 1, (c >> 2) & 1
                    wc = a[dy][dx] * zcs[dz]
                    word = gbuf[b, 0, pl.ds(p0 * 8 + c * 16, 16)]
                    f0c = lax.bitcast_convert_type(word << 16, jnp.float32)
                    f1c = lax.bitcast_convert_type(word & np.int32(-65536),
                                                   jnp.float32)
                    acc0 = acc0 + wc * f0c
                    acc1 = acc1 + wc * f1c
                fbuf[pl.ds((2 * l) * C + p0, 16)] = acc0
                fbuf[pl.ds((2 * l + 1) * C + p0, 16)] = acc1
                return carry

            lax.fori_loop(0, C // 16, body, 0, unroll=False)

        def chunk_body(ch, carry):
            row0 = base + ch * C
            for dim in range(3):
                pltpu.sync_copy(xct_hbm.at[pl.ds(dim * n_pts + row0, C)],
                                xbuf.at[pl.ds(dim * C, C)])
            idx_pass(0, 0)
            pending = {0: fire(0, 0)}
            for l in range(N_LEVELS):
                if l + 1 < N_LEVELS:
                    idx_pass(l + 1, (l + 1) % 2)
                    pending[l + 1] = fire(l + 1, (l + 1) % 2)
                for h in pending.pop(l):
                    h.wait()
                mac_pass(l, l % 2)
            pltpu.sync_copy(fbuf, out_hbm.at[pl.ds(row0 * 32, C * 32)])
            return carry

        lax.fori_loop(0, nch, chunk_body, 0, unroll=False)

    return encode


BN = 2048


def _sh16_rows(x, y, z):
    xy = x * y
    xz = x * z
    yz = y * z
    x2 = x * x
    y2 = y * y
    z2 = z * z
    return jnp.concatenate([
        0.28209479177387814 * jnp.ones_like(x),
        -0.48860251190291987 * y,
        0.48860251190291987 * z,
        -0.48860251190291987 * x,
        1.0925484305920792 * xy,
        -1.0925484305920792 * yz,
        0.94617469575755997 * z2 - 0.31539156525251999,
        -1.0925484305920792 * xz,
        0.54627421529603959 * (x2 - y2),
        0.59004358992664352 * y * (-3.0 * x2 + y2),
        2.8906114426405538 * xy * z,
        0.45704579946446572 * y * (1.0 - 5.0 * z2),
        0.3731763325901154 * z * (5.0 * z2 - 3.0),
        0.45704579946446572 * x * (1.0 - 5.0 * z2),
        1.4453057213202769 * z * (x2 - y2),
        0.59004358992664352 * x * (-x2 + 3.0 * y2),
    ], axis=0)


def _mlp_body(feats_ref, dt_ref, xw0t_ref, xb0c_ref, xw1at_ref, xb1ac_ref,
              xw1bt_ref, xb1bc_ref, dw0t_ref, db0c_ref, dw1t_ref, db1c_ref,
              dw2t_ref, db2c_ref, sigma_ref, rgbt_ref):
    feats = feats_ref[...]
    h = jnp.maximum(
        jnp.dot(xw0t_ref[...], feats, preferred_element_type=jnp.float32)
        + xb0c_ref[...], 0.0)
    f0 = (jnp.dot(xw1at_ref[...], h, preferred_element_type=jnp.float32)
          + xb1ac_ref[...])
    frest = (jnp.dot(xw1bt_ref[...], h, preferred_element_type=jnp.float32)
             + xb1bc_ref[...])
    sigma_ref[...] = jnp.log1p(jnp.exp(-jnp.abs(f0))) + jnp.maximum(f0, 0.0)
    dv = dt_ref[...]
    u = dv * 0.5 + 0.5
    v = u * 2.0 - 1.0
    sh = _sh16_rows(v[0:1, :], v[1:2, :], v[2:3, :])
    hd = jnp.concatenate([sh, frest], axis=0)
    h1 = jnp.maximum(
        jnp.dot(dw0t_ref[...], hd, preferred_element_type=jnp.float32)
        + db0c_ref[...], 0.0)
    h2 = jnp.maximum(
        jnp.dot(dw1t_ref[...], h1, preferred_element_type=jnp.float32)
        + db1c_ref[...], 0.0)
    out = (jnp.dot(dw2t_ref[...], h2, preferred_element_type=jnp.float32)
           + db2c_ref[...])
    rgbt_ref[...] = jax.nn.sigmoid(out)


def _full_spec(shape):
    nd = len(shape)
    return pl.BlockSpec(shape, lambda i: (0,) * nd)


def _mlp_call(feats_t, d_t, *weights):
    n = feats_t.shape[1]
    grid = (n // BN,)
    return pl.pallas_call(
        _mlp_body,
        grid=grid,
        in_specs=[
            pl.BlockSpec((32, BN), lambda i: (0, i)),
            pl.BlockSpec((3, BN), lambda i: (0, i)),
        ] + [_full_spec(w.shape) for w in weights],
        out_specs=[
            pl.BlockSpec((1, BN), lambda i: (0, i)),
            pl.BlockSpec((3, BN), lambda i: (0, i)),
        ],
        out_shape=[
            jax.ShapeDtypeStruct((1, n), jnp.float32),
            jax.ShapeDtypeStruct((3, n), jnp.float32),
        ],
    )(feats_t, d_t, *weights)


def kernel(x, d, hash_tables, xW0, xb0, xW1, xb1, dW0, db0, dW1, db1, dW2,
           db2):
    n = x.shape[0]
    xn = (x - AABB_LO) / (AABB_HI - AABB_LO) * 2.0 - 1.0
    xc = xn / 4.0 + 0.5
    xct = xc.T.reshape(-1)
    ptab = lax.bitcast_convert_type(
        hash_tables.astype(jnp.bfloat16), jnp.int32).reshape(-1)
    feats_cm = _make_encoder(n)(xct, ptab)
    feats_t = feats_cm.reshape(n // C, 32, C).transpose(1, 0, 2).reshape(32, n)
    sigma2d, rgbt = _mlp_call(
        feats_t, d.T,
        xW0.T, xb0[:, None],
        xW1[:, 0:1].T, xb1[0:1][:, None],
        xW1[:, 1:].T, xb1[1:][:, None],
        dW0.T, db0[:, None],
        dW1.T, db1[:, None],
        dW2.T, db2[:, None],
    )
    return sigma2d.reshape(n), rgbt.T
